# Initial kernel scaffold; baseline (speedup 1.0000x reference)
import jax
import jax.numpy as jnp
from jax.experimental import pallas as pl


def kernel(x, weight, gamma, beta, running_mean, running_var):
    raise NotImplementedError("write your pallas kernel here")


# speedup vs baseline: 1.1879x; 1.1879x over previous
import functools

import jax
import jax.numpy as jnp
from jax import lax
from jax.experimental import pallas as pl
from jax.experimental.pallas import tpu as pltpu


def _conv_bn_relu_kernel(x_ref, w_ref, b_ref, o_ref, *, H, W, Cin, Cout):
    M = H * W
    x = x_ref[0].astype(jnp.bfloat16)

    col = lax.broadcasted_iota(jnp.int32, (1, M), 1)
    wo = col % W
    mask_l = wo >= 1
    mask_r = wo <= W - 2

    acc = jnp.zeros((Cout, M), jnp.float32)
    for kh in range(3):
        for kw in range(3):
            s = (kh - 1) * W + (kw - 1)
            if s > 0:
                xs = jnp.concatenate(
                    [x[:, s:], jnp.zeros((Cin, s), jnp.bfloat16)], axis=1)
            elif s < 0:
                xs = jnp.concatenate(
                    [jnp.zeros((Cin, -s), jnp.bfloat16), x[:, :M + s]], axis=1)
            else:
                xs = x
            if kw == 0:
                xs = jnp.where(mask_l, xs, jnp.bfloat16(0))
            elif kw == 2:
                xs = jnp.where(mask_r, xs, jnp.bfloat16(0))
            acc = acc + jnp.dot(w_ref[kh * 3 + kw], xs,
                                preferred_element_type=jnp.float32)

    y = jnp.maximum(acc + b_ref[...], 0.0)
    o_ref[0] = y.astype(o_ref.dtype)


def kernel(x, weight, gamma, beta, running_mean, running_var):
    B, Cin, H, W = x.shape
    Cout, Cin_w, KH, KW = weight.shape
    assert (Cin_w, KH, KW) == (Cin, 3, 3)
    M = H * W

    inv = gamma / jnp.sqrt(running_var + 1e-5)
    shift = (beta - running_mean * inv).astype(jnp.float32)
    w_scaled = weight * inv[:, None, None, None]
    w_taps = jnp.transpose(w_scaled, (2, 3, 0, 1)).reshape(
        KH * KW, Cout, Cin).astype(jnp.bfloat16)

    x_flat = x.reshape(B, Cin, M)

    out = pl.pallas_call(
        functools.partial(_conv_bn_relu_kernel, H=H, W=W, Cin=Cin, Cout=Cout),
        out_shape=jax.ShapeDtypeStruct((B, Cout, M), x.dtype),
        grid=(B,),
        in_specs=[
            pl.BlockSpec((1, Cin, M), lambda b: (b, 0, 0)),
            pl.BlockSpec((KH * KW, Cout, Cin), lambda b: (0, 0, 0)),
            pl.BlockSpec((Cout, 1), lambda b: (0, 0)),
        ],
        out_specs=pl.BlockSpec((1, Cout, M), lambda b: (b, 0, 0)),
        compiler_params=pltpu.CompilerParams(
            dimension_semantics=("parallel",)),
    )(x_flat, w_taps, shift.reshape(Cout, 1))

    return out.reshape(B, Cout, H, W)
